# Initial kernel scaffold; baseline (speedup 1.0000x reference)
import jax
import jax.numpy as jnp
from jax.experimental import pallas as pl


def kernel(x, embed):
    raise NotImplementedError("write your pallas kernel here")


# speedup vs baseline: 1.8653x; 1.8653x over previous
import functools

import jax
import jax.numpy as jnp
from jax import lax
from jax.experimental import pallas as pl

DIM = 256
K = 1024
TN = 512


def _vq_body(x_ref, e_ref, q_ref, idx_ref):
    x = x_ref[...]
    e = e_ref[...]
    s = lax.dot_general(x, e, (((1,), (1,)), ((), ())),
                        preferred_element_type=jnp.float32)
    xnorm = jnp.sum(x * x, axis=1, keepdims=True)
    enorm = jnp.sum(e * e, axis=1)[None, :]
    dist = -(xnorm - 2.0 * s + enorm)
    iota = lax.broadcasted_iota(jnp.int32, (TN, K), 1)
    m = jnp.max(dist, axis=1, keepdims=True)
    idx = jnp.min(jnp.where(dist == m, iota, K), axis=1)
    idx_ref[...] = idx
    onehot = (iota == idx[:, None]).astype(jnp.float32)
    q_ref[...] = lax.dot_general(onehot, e, (((1,), (0,)), ((), ())),
                                 preferred_element_type=jnp.float32)


@jax.jit
def _vq(flat, embed):
    n = flat.shape[0]
    grid = (n // TN,)
    return pl.pallas_call(
        _vq_body,
        grid=grid,
        in_specs=[
            pl.BlockSpec((TN, DIM), lambda i: (i, 0)),
            pl.BlockSpec((K, DIM), lambda i: (0, 0)),
        ],
        out_specs=[
            pl.BlockSpec((TN, DIM), lambda i: (i, 0)),
            pl.BlockSpec((TN,), lambda i: (i,)),
        ],
        out_shape=[
            jax.ShapeDtypeStruct((n, DIM), jnp.float32),
            jax.ShapeDtypeStruct((n,), jnp.int32),
        ],
    )(flat, embed)


def kernel(x, embed):
    shape = x.shape
    flat = x.reshape(-1, shape[-1])
    quantize, idx = _vq(flat, embed)
    return quantize.reshape(shape), idx.reshape(shape[:-1])
